# baseline (device time: 52615 ns/iter reference)
import jax
import jax.numpy as jnp
from jax import lax
from jax.experimental import pallas as pl
from jax.experimental.pallas import tpu as pltpu

CAP = 1152
CH = 64
NCH = CAP // CH


def kernel(ids, E):
    T = ids.shape[0]
    V, D = E.shape

    my_x = lax.axis_index("x")
    local_idx = jnp.clip(ids - my_x * V, 0, V - 1).astype(jnp.int32)
    owned = (ids >= my_x * V) & (ids < (my_x + 1) * V)
    own_i32 = owned.astype(jnp.int32)

    _, cidx_full = lax.sort_key_val(1 - own_i32, local_idx, is_stable=True)
    cidx = cidx_full[:CAP]

    km = jnp.cumsum(own_i32) - 1
    kt = jnp.cumsum(1 - own_i32) - 1
    pos = jnp.where(owned, km, CAP + kt).astype(jnp.int32)[:, None]

    def body(cidx_ref, pos_ref, E_ref, out_ref,
             compact_ref, comm_ref, gather_sems, rs_sem, rr_sem):
        x = lax.axis_index("x")
        y = lax.axis_index("y")
        z = lax.axis_index("z")
        partner = (1 - x, y, z)

        def issue_chunk(c):
            def f(i, _):
                pltpu.make_async_copy(
                    E_ref.at[pl.ds(cidx_ref[i], 1), :],
                    compact_ref.at[pl.ds(i, 1), :],
                    gather_sems.at[c],
                ).start()
                return ()
            lax.fori_loop(c * CH, (c + 1) * CH, f, (), unroll=8)

        def process_chunk(c):
            def g(i, _):
                pltpu.make_async_copy(
                    E_ref.at[pl.ds(0, 1), :],
                    compact_ref.at[pl.ds(0, 1), :],
                    gather_sems.at[c],
                ).wait()
                return ()
            lax.fori_loop(0, CH, g, (), unroll=8)

            sl = pl.ds(c * CH, CH)
            comm_ref[sl, :] = compact_ref[sl, :].astype(jnp.bfloat16)
            pltpu.make_async_remote_copy(
                src_ref=comm_ref.at[sl, :],
                dst_ref=comm_ref.at[pl.ds(CAP + c * CH, CH), :],
                send_sem=rs_sem,
                recv_sem=rr_sem,
                device_id=partner,
                device_id_type=pl.DeviceIdType.MESH,
            ).start()

        barrier = pltpu.get_barrier_semaphore()
        pl.semaphore_signal(
            barrier, inc=1, device_id=partner,
            device_id_type=pl.DeviceIdType.MESH,
        )
        issue_chunk(0)
        pl.semaphore_wait(barrier, 1)
        for c in range(1, NCH):
            issue_chunk(c)
            process_chunk(c - 1)
        process_chunk(NCH - 1)

        iota = lax.broadcasted_iota(jnp.int32, (T, 2 * CAP), 1)
        s = (iota == pos_ref[...]).astype(jnp.bfloat16)
        m1 = jnp.dot(s[:, :CAP], comm_ref[pl.ds(0, CAP), :],
                     preferred_element_type=jnp.float32)

        chunk_rdma = pltpu.make_async_remote_copy(
            src_ref=comm_ref.at[pl.ds(0, CH), :],
            dst_ref=comm_ref.at[pl.ds(CAP, CH), :],
            send_sem=rs_sem,
            recv_sem=rr_sem,
            device_id=partner,
            device_id_type=pl.DeviceIdType.MESH,
        )
        for _ in range(NCH):
            chunk_rdma.wait_recv()
        for _ in range(NCH):
            chunk_rdma.wait_send()

        m2 = jnp.dot(s[:, CAP:], comm_ref[pl.ds(CAP, CAP), :],
                     preferred_element_type=jnp.float32)
        out_ref[...] = (m1 + m2).astype(jnp.bfloat16)

    return pl.pallas_call(
        body,
        out_shape=jax.ShapeDtypeStruct((T, D), jnp.bfloat16),
        in_specs=[
            pl.BlockSpec(memory_space=pltpu.SMEM),
            pl.BlockSpec(memory_space=pltpu.VMEM),
            pl.BlockSpec(memory_space=pltpu.MemorySpace.HBM),
        ],
        out_specs=pl.BlockSpec(memory_space=pltpu.VMEM),
        scratch_shapes=[
            pltpu.VMEM((CAP, D), jnp.float32),
            pltpu.VMEM((2 * CAP, D), jnp.bfloat16),
            pltpu.SemaphoreType.DMA((NCH,)),
            pltpu.SemaphoreType.DMA,
            pltpu.SemaphoreType.DMA,
        ],
        compiler_params=pltpu.CompilerParams(collective_id=0),
    )(cidx, pos, E)


# device time: 52570 ns/iter; 1.0009x vs baseline; 1.0009x over previous
import jax
import jax.numpy as jnp
from jax import lax
from jax.experimental import pallas as pl
from jax.experimental.pallas import tpu as pltpu

CAP = 1152
CH = 64
NCH = CAP // CH


def kernel(ids, E):
    T = ids.shape[0]
    V, D = E.shape

    my_x = lax.axis_index("x")
    local_idx = jnp.clip(ids - my_x * V, 0, V - 1).astype(jnp.int32)
    owned = (ids >= my_x * V) & (ids < (my_x + 1) * V)
    own_i32 = owned.astype(jnp.int32)

    _, cidx_full = lax.sort_key_val(1 - own_i32, local_idx, is_stable=True)
    cidx = cidx_full[:CAP]

    km = jnp.cumsum(own_i32) - 1
    kt = jnp.cumsum(1 - own_i32) - 1
    pos = jnp.where(owned, km, CAP + kt).astype(jnp.int32)[:, None]

    def body(cidx_ref, pos_ref, E_ref, out_ref,
             compact_ref, comm_ref, gather_sems, rs_sem, rr_sem):
        x = lax.axis_index("x")
        y = lax.axis_index("y")
        z = lax.axis_index("z")
        partner = (1 - x, y, z)

        def issue_chunk(c):
            def f(i, _):
                pltpu.make_async_copy(
                    E_ref.at[pl.ds(cidx_ref[i], 1), :],
                    compact_ref.at[pl.ds(i, 1), :],
                    gather_sems.at[c],
                ).start()
                return ()
            lax.fori_loop(c * CH, (c + 1) * CH, f, (), unroll=16)

        def process_chunk(c):
            def g(i, _):
                pltpu.make_async_copy(
                    E_ref.at[pl.ds(0, 1), :],
                    compact_ref.at[pl.ds(0, 1), :],
                    gather_sems.at[c],
                ).wait()
                return ()
            lax.fori_loop(0, CH, g, (), unroll=16)

            sl = pl.ds(c * CH, CH)
            comm_ref[sl, :] = compact_ref[sl, :].astype(jnp.bfloat16)
            pltpu.make_async_remote_copy(
                src_ref=comm_ref.at[sl, :],
                dst_ref=comm_ref.at[pl.ds(CAP + c * CH, CH), :],
                send_sem=rs_sem,
                recv_sem=rr_sem,
                device_id=partner,
                device_id_type=pl.DeviceIdType.MESH,
            ).start()

        barrier = pltpu.get_barrier_semaphore()
        pl.semaphore_signal(
            barrier, inc=1, device_id=partner,
            device_id_type=pl.DeviceIdType.MESH,
        )
        issue_chunk(0)
        pl.semaphore_wait(barrier, 1)
        for c in range(1, NCH):
            issue_chunk(c)
            process_chunk(c - 1)
        process_chunk(NCH - 1)

        iota = lax.broadcasted_iota(jnp.int32, (T, 2 * CAP), 1)
        s = (iota == pos_ref[...]).astype(jnp.bfloat16)
        m1 = jnp.dot(s[:, :CAP], comm_ref[pl.ds(0, CAP), :],
                     preferred_element_type=jnp.float32)

        chunk_rdma = pltpu.make_async_remote_copy(
            src_ref=comm_ref.at[pl.ds(0, CH), :],
            dst_ref=comm_ref.at[pl.ds(CAP, CH), :],
            send_sem=rs_sem,
            recv_sem=rr_sem,
            device_id=partner,
            device_id_type=pl.DeviceIdType.MESH,
        )
        for _ in range(NCH):
            chunk_rdma.wait_recv()
        for _ in range(NCH):
            chunk_rdma.wait_send()

        m2 = jnp.dot(s[:, CAP:], comm_ref[pl.ds(CAP, CAP), :],
                     preferred_element_type=jnp.float32)
        out_ref[...] = (m1 + m2).astype(jnp.bfloat16)

    return pl.pallas_call(
        body,
        out_shape=jax.ShapeDtypeStruct((T, D), jnp.bfloat16),
        in_specs=[
            pl.BlockSpec(memory_space=pltpu.SMEM),
            pl.BlockSpec(memory_space=pltpu.VMEM),
            pl.BlockSpec(memory_space=pltpu.MemorySpace.HBM),
        ],
        out_specs=pl.BlockSpec(memory_space=pltpu.VMEM),
        scratch_shapes=[
            pltpu.VMEM((CAP, D), jnp.float32),
            pltpu.VMEM((2 * CAP, D), jnp.bfloat16),
            pltpu.SemaphoreType.DMA((NCH,)),
            pltpu.SemaphoreType.DMA,
            pltpu.SemaphoreType.DMA,
        ],
        compiler_params=pltpu.CompilerParams(collective_id=0),
    )(cidx, pos, E)
